# baseline (speedup 1.0000x reference)
import jax
import jax.numpy as jnp
from jax import lax
from jax.experimental import pallas as pl
from jax.experimental.pallas import tpu as pltpu
from jax.experimental.pallas import tpu_sc as plsc

N = 10000
E = 320000
D = 128
MIN_NORM = 1e-15

NC = 2
NS = 16
NW = NC * NS
EPT = E // NW
K = 80
NCHUNK = EPT // K
NPAIR = (NCHUNK - 1) // 2
SK = 128
SNCHUNK = E // SK
SCHUNK_BASE = SNCHUNK // NS
SCHUNK_REM = SNCHUNK - SCHUNK_BASE * NS
HALF = 5120
ACC_ROWS = 5248

_F32 = jnp.float32
_I32 = jnp.int32


def _stage1_body(x_ref, w_ref, t_ref):
    t_ref[:, :D] = x_ref[...]
    t_ref[:, D:] = jnp.dot(x_ref[...], w_ref[...],
                           preferred_element_type=jnp.float32)


def _stage1(x, w1a):
    bm = 2000
    return pl.pallas_call(
        _stage1_body,
        grid=(N // bm,),
        in_specs=[
            pl.BlockSpec((bm, D), lambda i: (i, 0)),
            pl.BlockSpec((D, D), lambda i: (0, 0)),
        ],
        out_specs=pl.BlockSpec((bm, 2 * D), lambda i: (i, 0)),
        out_shape=jax.ShapeDtypeStruct((N, 2 * D), jnp.float32),
    )(x, w1a)


def _stage3_body(agg_ref, x_ref, wn1_ref, bn1_ref, wn2_ref, bn2_ref, o_ref):
    agg = agg_ref[...] * 0.01
    h2 = jnp.maximum(
        jnp.dot(agg, wn1_ref[...], preferred_element_type=jnp.float32)
        + bn1_ref[...], 0.0)
    s = (jnp.dot(h2, wn2_ref[...], preferred_element_type=jnp.float32)
         + bn2_ref[...])
    x = x_ref[...]
    u2 = jnp.sum(s * s, axis=-1, keepdims=True)
    u_norm = jnp.sqrt(jnp.clip(u2, MIN_NORM, None))
    x2 = jnp.sum(x * x, axis=-1, keepdims=True)
    lam = 2.0 / jnp.clip(1.0 - x2, MIN_NORM, None)
    second = jnp.tanh(0.5 * lam * u_norm) / u_norm * s
    y2 = jnp.sum(second * second, axis=-1, keepdims=True)
    xy = jnp.sum(x * second, axis=-1, keepdims=True)
    num = (1.0 + 2.0 * xy + y2) * x + (1.0 - x2) * second
    den = jnp.clip(1.0 + 2.0 * xy + x2 * y2, MIN_NORM, None)
    res = num / den
    rn = jnp.sqrt(jnp.clip(jnp.sum(res * res, axis=-1, keepdims=True),
                           MIN_NORM, None))
    maxnorm = 1.0 - 1e-5
    o_ref[...] = jnp.where(rn > maxnorm, res / rn * maxnorm, res)


def _stage3(agg, x, wn1b, bn1, wn2, bn2):
    bm = 2000
    return pl.pallas_call(
        _stage3_body,
        grid=(N // bm,),
        in_specs=[
            pl.BlockSpec((bm, D), lambda i: (i, 0)),
            pl.BlockSpec((bm, D), lambda i: (i, 0)),
            pl.BlockSpec((D, D), lambda i: (0, 0)),
            pl.BlockSpec((D,), lambda i: (0,)),
            pl.BlockSpec((D, D), lambda i: (0, 0)),
            pl.BlockSpec((D,), lambda i: (0,)),
        ],
        out_specs=pl.BlockSpec((bm, D), lambda i: (i, 0)),
        out_shape=jax.ShapeDtypeStruct((N, D), jnp.float32),
    )(agg, x, wn1b, bn1, wn2, bn2)


def _sqrt16(v):
    i = plsc.bitcast(v, _I32)
    i = jnp.int32(0x5F3759DF) - (i >> 1)
    y = plsc.bitcast(i, _F32)
    y = y * (1.5 - 0.5 * v * y * y)
    y = y * (1.5 - 0.5 * v * y * y)
    y = y * (1.5 - 0.5 * v * y * y)
    return v * y


def _artanh_ratio16(sn):
    z = jnp.minimum(sn, 1.0 - 1e-7)
    zz = z * z
    small = 1.0 + zz * (1.0 / 3.0 + zz * (0.2 + zz * (1.0 / 7.0
                                                      + zz * (1.0 / 9.0))))
    w = (1.0 + z) / (1.0 - z)
    iw = plsc.bitcast(w, _I32)
    ef = ((iw >> 23) - 127).astype(_F32)
    m = plsc.bitcast((iw & jnp.int32(0x007FFFFF)) | jnp.int32(0x3F800000),
                     _F32)
    s = (m - 1.0) / (m + 1.0)
    s2 = s * s
    log2m = s * (2.885390082 + s2 * (0.961796694 + s2 * (
        0.577078016 + s2 * (0.412198583 + s2 * 0.320598898))))
    big = 0.34657359028 * (ef + log2m) / sn
    return jnp.where(z < 0.25, small, big)


def _edges_body(t_hbm, row_hbm, col_hbm, dist_hbm, mask_hbm, wv_hbm,
                spill_hbm,
                idxr0, idxc0, dbuf0, mbuf0, tr0, tc0,
                idxr1, idxc1, dbuf1, mbuf1, tr1, tc1,
                vbuf, wv, sem0, sem1):
    cid = lax.axis_index("c")
    sid = lax.axis_index("s")
    w = cid * NS + sid
    ebase = w * EPT

    pltpu.sync_copy(wv_hbm, wv)
    b2 = wv[3, pl.ds(0, 16)][0]
    lanes = lax.iota(_I32, 16)
    zero16 = jnp.zeros((16,), _F32)

    slots = ((idxr0, idxc0, dbuf0, mbuf0, tr0, tc0, sem0),
             (idxr1, idxc1, dbuf1, mbuf1, tr1, tc1, sem1))

    def _issue(t, s):
        idxr, idxc, dbuf, mbuf, tr, tc, sem = s
        base = ebase + t * K
        pltpu.sync_copy(row_hbm.at[pl.ds(base, K)], idxr)
        pltpu.sync_copy(col_hbm.at[pl.ds(base, K)], idxc)
        pltpu.async_copy(dist_hbm.at[pl.ds(base, K)], dbuf.at[pl.ds(0, K)], sem)
        pltpu.async_copy(mask_hbm.at[pl.ds(base, K)], mbuf.at[pl.ds(0, K)], sem)
        pltpu.async_copy(t_hbm.at[idxr], tr, sem)
        pltpu.async_copy(t_hbm.at[idxc], tc, sem)

    def _wait(s):
        idxr, idxc, dbuf, mbuf, tr, tc, sem = s
        pltpu.make_async_copy(dist_hbm.at[pl.ds(0, K)], dbuf.at[pl.ds(0, K)], sem).wait()
        pltpu.make_async_copy(mask_hbm.at[pl.ds(0, K)], mbuf.at[pl.ds(0, K)], sem).wait()
        pltpu.make_async_copy(t_hbm.at[idxr], tr, sem).wait()
        pltpu.make_async_copy(t_hbm.at[idxc], tc, sem).wait()

    def _rcp(x):
        r = lax.bitcast_convert_type(
            jnp.int32(0x7EF311C3) - lax.bitcast_convert_type(x, _I32),
            _F32)
        r = r * (2.0 - x * r)
        r = r * (2.0 - x * r)
        r = r * (2.0 - x * r)
        return r

    NB = D // 16
    wdv = [wv[0, pl.ds(16 * i, 16)] for i in range(NB)]
    b1v = [wv[1, pl.ds(16 * i, 16)] for i in range(NB)]
    w2v = [wv[2, pl.ds(16 * i, 16)] for i in range(NB)]

    def _compute(t, s):
        idxr, idxc, dbuf, mbuf, tr, tc, sem = s
        base = ebase + t * K

        def _group(g, _):
            dv16 = dbuf[pl.ds(g * 8, 16)]
            mv16 = mbuf[pl.ds(g * 8, 16)]
            for j in range(8):
                e = g * 8 + j
                ar = [tr[e, pl.ds(16 * i, 16)] for i in range(NB)]
                br = [tc[e, pl.ds(16 * i, 16)] for i in range(NB)]
                sxy = zero16
                sx2 = zero16
                sy2 = zero16
                for i in range(NB):
                    sxy = sxy + ar[i] * br[i]
                    sx2 = sx2 + ar[i] * ar[i]
                    sy2 = sy2 + br[i] * br[i]
                xy = jnp.sum(sxy)
                x2 = jnp.sum(sx2)
                y2 = jnp.sum(sy2)

                A = 2.0 * xy - 1.0 - y2
                B = 1.0 - x2
                den = jnp.maximum(1.0 - 2.0 * xy + x2 * y2, MIN_NORM)
                rden = _rcp(den)
                sn2 = jnp.maximum(
                    (A * A * x2 + 2.0 * A * B * xy + B * B * y2)
                    * (rden * rden), MIN_NORM)
                ii = jnp.int32(0x5F3759DF) - (
                    lax.bitcast_convert_type(sn2, _I32) >> 1)
                yh = lax.bitcast_convert_type(ii, _F32)
                yh = yh * (1.5 - 0.5 * sn2 * yh * yh)
                yh = yh * (1.5 - 0.5 * sn2 * yh * yh)
                yh = yh * (1.5 - 0.5 * sn2 * yh * yh)
                sn = sn2 * yh
                z = jnp.minimum(sn, 1.0 - 1e-7)
                zz = z * z
                small = 1.0 + zz * (1.0 / 3.0 + zz * (
                    0.2 + zz * (1.0 / 7.0 + zz * (1.0 / 9.0))))
                ww = (1.0 + z) * _rcp(1.0 - z)
                iw = lax.bitcast_convert_type(ww, _I32)
                ef = ((iw >> 23) - 127).astype(_F32)
                mm = lax.bitcast_convert_type(
                    (iw & jnp.int32(0x007FFFFF)) | jnp.int32(0x3F800000),
                    _F32)
                ss = (mm - 1.0) * _rcp(mm + 1.0)
                s2 = ss * ss
                log2m = ss * (2.885390082 + s2 * (0.961796694 + s2 * (
                    0.577078016 + s2 * (0.412198583 + s2 * 0.320598898))))
                big = 0.34657359028 * (ef + log2m) * yh
                ratio = jnp.where(z < 0.25, small, big)
                kd = B * ratio * rden
                p = kd * A
                q = kd * B
                dd = dv16[j]

                la = zero16
                lb = zero16
                for i in range(NB):
                    gr = tr[e, pl.ds(D + 16 * i, 16)]
                    gc = tc[e, pl.ds(D + 16 * i, 16)]
                    pre = p * gr + q * gc + (dd * wdv[i] + b1v[i])
                    sig = 1.0 / (1.0 + jnp.exp(-pre))
                    if i % 2 == 0:
                        la = la + (pre * sig) * w2v[i]
                    else:
                        lb = lb + (pre * sig) * w2v[i]
                logit = jnp.sum(la + lb) + b2
                lvec = jnp.full((16,), logit, _F32)
                att = mv16[j] / (1.0 + jnp.exp(-lvec))
                w1 = att * p
                w2 = att * q
                for i in range(NB):
                    a2 = tr[e, pl.ds(16 * i, 16)]
                    b2r = tc[e, pl.ds(16 * i, 16)]
                    vbuf[e, pl.ds(16 * i, 16)] = w1 * a2 + w2 * b2r
            return 0

        lax.fori_loop(0, K // 8, _group, 0)
        pltpu.sync_copy(vbuf, spill_hbm.at[pl.ds(base, K)])

    _issue(0, slots[0])

    def _pair(tb, _):
        t0 = 2 * tb
        _issue(t0 + 1, slots[1])
        _wait(slots[0])
        _compute(t0, slots[0])
        _issue(t0 + 2, slots[0])
        _wait(slots[1])
        _compute(t0 + 1, slots[1])
        return 0

    lax.fori_loop(0, NPAIR, _pair, 0)
    _wait(slots[0])
    _compute(NCHUNK - 1, slots[0])


def _edges_sc(table, row, col, dist, emask, wvec):
    mesh = plsc.VectorSubcoreMesh(core_axis_name="c", subcore_axis_name="s")
    slot = [
        pltpu.VMEM((K,), _I32),
        pltpu.VMEM((K,), _I32),
        pltpu.VMEM((K + 16,), _F32),
        pltpu.VMEM((K + 16,), _F32),
        pltpu.VMEM((K, 2 * D), _F32),
        pltpu.VMEM((K, 2 * D), _F32),
    ]
    return pl.kernel(
        _edges_body,
        out_type=jax.ShapeDtypeStruct((E, D), jnp.float32),
        mesh=mesh,
        compiler_params=pltpu.CompilerParams(needs_layout_passes=False,
                                             use_tc_tiling_on_sc=False),
        scratch_types=slot + slot + [
            pltpu.VMEM((K, D), _F32),
            pltpu.VMEM((4, D), _F32),
            pltpu.SemaphoreType.DMA,
            pltpu.SemaphoreType.DMA,
        ],
    )(table, row, col, dist, emask, wvec)


def _scatter_body(spill_hbm, row_hbm, out_hbm, idxr, idxl, vbuf, acc, sem):
    cid = lax.axis_index("c")
    sid = lax.axis_index("s")
    lo = cid * HALF

    zero16 = jnp.zeros((16,), _F32)

    def _zrow(i, _):
        for j in range(D // 16):
            vbuf[i, pl.ds(16 * j, 16)] = zero16
        return 0

    lax.fori_loop(0, SK, _zrow, 0)
    for j, h in ((0, 128), (128, 128), (256, 72)):
        pltpu.sync_copy(vbuf.at[pl.ds(0, h)],
                        acc.at[pl.ds(sid * 328 + j, h)])
    plsc.subcore_barrier()

    nt = jnp.where(sid < SCHUNK_REM, SCHUNK_BASE + 1, SCHUNK_BASE)
    start = sid * SCHUNK_BASE + jnp.minimum(sid, SCHUNK_REM)

    def _chunk(t, _):
        base = (start + t) * SK
        pltpu.sync_copy(row_hbm.at[pl.ds(base, SK)], idxr)
        c1 = pltpu.async_copy(spill_hbm.at[pl.ds(base, SK)], vbuf, sem)
        for g in range(SK // 16):
            r = idxr[pl.ds(g * 16, 16)]
            rl = r - lo
            ok = (rl >= 0) & (rl < HALF)
            rl = jnp.where(ok, rl, HALF)
            idxl[pl.ds(g * 16, 16)] = rl
        c1.wait()
        pltpu.sync_copy(vbuf, acc.at[idxl], add=True)
        return 0

    lax.fori_loop(0, nt, _chunk, 0)
    plsc.subcore_barrier()
    pltpu.sync_copy(acc.at[pl.ds(sid * 320, 320)],
                    out_hbm.at[cid, pl.ds(sid * 320, 320)])


def _scatter_sc(spill, row):
    mesh = plsc.VectorSubcoreMesh(core_axis_name="c", subcore_axis_name="s")
    return pl.kernel(
        _scatter_body,
        out_type=jax.ShapeDtypeStruct((NC, HALF, D), jnp.float32),
        mesh=mesh,
        compiler_params=pltpu.CompilerParams(needs_layout_passes=False,
                                             use_tc_tiling_on_sc=False),
        scratch_types=[
            pltpu.VMEM((SK,), _I32),
            pltpu.VMEM((SK,), _I32),
            pltpu.VMEM((SK, D), _F32),
            pltpu.VMEM_SHARED((ACC_ROWS, D), _F32),
            pltpu.SemaphoreType.DMA,
        ],
    )(spill, row)


def kernel(x, distances, edges, node_mask, edge_mask, W_att1, b_att1,
           W_att2, b_att2, W_n1, b_n1, W_n2, b_n2):
    table = _stage1(x, W_att1[:D])
    wvec = jnp.stack([W_att1[2 * D], b_att1, W_att2[:, 0],
                      jnp.full((D,), b_att2[0], jnp.float32)])
    row = edges[0].astype(jnp.int32)
    col = edges[1].astype(jnp.int32)
    spill = _edges_sc(table, row, col, distances[:, 0], edge_mask[:, 0],
                      wvec)
    parts = _scatter_sc(spill, row)
    agg = jnp.concatenate([parts[0], parts[1]], axis=0)[:N]
    return _stage3(agg, x, W_n1[D:], b_n1, W_n2, b_n2)


# speedup vs baseline: 1.6908x; 1.6908x over previous
import jax
import jax.numpy as jnp
from jax import lax
from jax.experimental import pallas as pl
from jax.experimental.pallas import tpu as pltpu
from jax.experimental.pallas import tpu_sc as plsc

N = 10000
E = 320000
D = 128
MIN_NORM = 1e-15

NC = 2
NS = 16
NW = NC * NS
EPT = E // NW
K = 80
NCHUNK = EPT // K
NPAIR = (NCHUNK - 1) // 2
SK = 128
SNCHUNK = E // SK
SCHUNK_BASE = SNCHUNK // NS
SCHUNK_REM = SNCHUNK - SCHUNK_BASE * NS
HALF = 5120
ACC_ROWS = 5248

_F32 = jnp.float32
_I32 = jnp.int32


def _stage1_body(x_ref, w_ref, t_ref):
    t_ref[:, :D] = x_ref[...]
    t_ref[:, D:] = jnp.dot(x_ref[...], w_ref[...],
                           preferred_element_type=jnp.float32)


def _stage1(x, w1a):
    bm = 2000
    return pl.pallas_call(
        _stage1_body,
        grid=(N // bm,),
        in_specs=[
            pl.BlockSpec((bm, D), lambda i: (i, 0)),
            pl.BlockSpec((D, D), lambda i: (0, 0)),
        ],
        out_specs=pl.BlockSpec((bm, 2 * D), lambda i: (i, 0)),
        out_shape=jax.ShapeDtypeStruct((N, 2 * D), jnp.float32),
    )(x, w1a)


def _stage3_body(agg_ref, x_ref, wn1_ref, bn1_ref, wn2_ref, bn2_ref, o_ref):
    agg = agg_ref[...] * 0.01
    h2 = jnp.maximum(
        jnp.dot(agg, wn1_ref[...], preferred_element_type=jnp.float32)
        + bn1_ref[...], 0.0)
    s = (jnp.dot(h2, wn2_ref[...], preferred_element_type=jnp.float32)
         + bn2_ref[...])
    x = x_ref[...]
    u2 = jnp.sum(s * s, axis=-1, keepdims=True)
    u_norm = jnp.sqrt(jnp.clip(u2, MIN_NORM, None))
    x2 = jnp.sum(x * x, axis=-1, keepdims=True)
    lam = 2.0 / jnp.clip(1.0 - x2, MIN_NORM, None)
    second = jnp.tanh(0.5 * lam * u_norm) / u_norm * s
    y2 = jnp.sum(second * second, axis=-1, keepdims=True)
    xy = jnp.sum(x * second, axis=-1, keepdims=True)
    num = (1.0 + 2.0 * xy + y2) * x + (1.0 - x2) * second
    den = jnp.clip(1.0 + 2.0 * xy + x2 * y2, MIN_NORM, None)
    res = num / den
    rn = jnp.sqrt(jnp.clip(jnp.sum(res * res, axis=-1, keepdims=True),
                           MIN_NORM, None))
    maxnorm = 1.0 - 1e-5
    o_ref[...] = jnp.where(rn > maxnorm, res / rn * maxnorm, res)


def _stage3(agg, x, wn1b, bn1, wn2, bn2):
    bm = 2000
    return pl.pallas_call(
        _stage3_body,
        grid=(N // bm,),
        in_specs=[
            pl.BlockSpec((bm, D), lambda i: (i, 0)),
            pl.BlockSpec((bm, D), lambda i: (i, 0)),
            pl.BlockSpec((D, D), lambda i: (0, 0)),
            pl.BlockSpec((D,), lambda i: (0,)),
            pl.BlockSpec((D, D), lambda i: (0, 0)),
            pl.BlockSpec((D,), lambda i: (0,)),
        ],
        out_specs=pl.BlockSpec((bm, D), lambda i: (i, 0)),
        out_shape=jax.ShapeDtypeStruct((N, D), jnp.float32),
    )(agg, x, wn1b, bn1, wn2, bn2)


def _sqrt16(v):
    i = plsc.bitcast(v, _I32)
    i = jnp.int32(0x5F3759DF) - (i >> 1)
    y = plsc.bitcast(i, _F32)
    y = y * (1.5 - 0.5 * v * y * y)
    y = y * (1.5 - 0.5 * v * y * y)
    y = y * (1.5 - 0.5 * v * y * y)
    return v * y


def _artanh_ratio16(sn):
    z = jnp.minimum(sn, 1.0 - 1e-7)
    zz = z * z
    small = 1.0 + zz * (1.0 / 3.0 + zz * (0.2 + zz * (1.0 / 7.0
                                                      + zz * (1.0 / 9.0))))
    w = (1.0 + z) / (1.0 - z)
    iw = plsc.bitcast(w, _I32)
    ef = ((iw >> 23) - 127).astype(_F32)
    m = plsc.bitcast((iw & jnp.int32(0x007FFFFF)) | jnp.int32(0x3F800000),
                     _F32)
    s = (m - 1.0) / (m + 1.0)
    s2 = s * s
    log2m = s * (2.885390082 + s2 * (0.961796694 + s2 * (
        0.577078016 + s2 * (0.412198583 + s2 * 0.320598898))))
    big = 0.34657359028 * (ef + log2m) / sn
    return jnp.where(z < 0.25, small, big)


def _edges_body(t_hbm, row_hbm, col_hbm, dist_hbm, mask_hbm, wv_hbm,
                spill_hbm,
                idxr0, idxc0, dbuf0, mbuf0, tr0, tc0,
                idxr1, idxc1, dbuf1, mbuf1, tr1, tc1,
                vbuf, wv, sem0, sem1):
    cid = lax.axis_index("c")
    sid = lax.axis_index("s")
    w = cid * NS + sid
    ebase = w * EPT

    pltpu.sync_copy(wv_hbm, wv)
    b2 = wv[3, pl.ds(0, 16)][0]
    lanes = lax.iota(_I32, 16)
    zero16 = jnp.zeros((16,), _F32)

    slots = ((idxr0, idxc0, dbuf0, mbuf0, tr0, tc0, sem0),
             (idxr1, idxc1, dbuf1, mbuf1, tr1, tc1, sem1))

    def _issue(t, s):
        idxr, idxc, dbuf, mbuf, tr, tc, sem = s
        base = ebase + t * K
        pltpu.sync_copy(row_hbm.at[pl.ds(base, K)], idxr)
        pltpu.sync_copy(col_hbm.at[pl.ds(base, K)], idxc)
        pltpu.async_copy(dist_hbm.at[pl.ds(base, K)], dbuf, sem)
        pltpu.async_copy(mask_hbm.at[pl.ds(base, K)], mbuf, sem)
        pltpu.async_copy(t_hbm.at[idxr], tr, sem)
        pltpu.async_copy(t_hbm.at[idxc], tc, sem)

    def _wait(s):
        idxr, idxc, dbuf, mbuf, tr, tc, sem = s
        pltpu.make_async_copy(dist_hbm.at[pl.ds(0, K)], dbuf, sem).wait()
        pltpu.make_async_copy(mask_hbm.at[pl.ds(0, K)], mbuf, sem).wait()
        pltpu.make_async_copy(t_hbm.at[idxr], tr, sem).wait()
        pltpu.make_async_copy(t_hbm.at[idxc], tc, sem).wait()

    def _compute(t, s):
        idxr, idxc, dbuf, mbuf, tr, tc, sem = s
        base = ebase + t * K

        def _group(g, _):
            rows = lanes + g * 16

            def _adot(d16, carry):
                c = list(carry)
                d0 = d16 * 16
                av = [plsc.load_gather(
                    tr, [rows, jnp.full((16,), d0 + j, _I32)])
                    for j in range(16)]
                bv = [plsc.load_gather(
                    tc, [rows, jnp.full((16,), d0 + j, _I32)])
                    for j in range(16)]
                for j in range(16):
                    k = 3 * (j % 4)
                    c[k] = c[k] + av[j] * bv[j]
                    c[k + 1] = c[k + 1] + av[j] * av[j]
                    c[k + 2] = c[k + 2] + bv[j] * bv[j]
                return tuple(c)

            acc12 = lax.fori_loop(0, D // 16, _adot, (zero16,) * 12)
            xy = acc12[0] + acc12[3] + acc12[6] + acc12[9]
            x2 = acc12[1] + acc12[4] + acc12[7] + acc12[10]
            y2 = acc12[2] + acc12[5] + acc12[8] + acc12[11]

            A = 2.0 * xy - 1.0 - y2
            B = 1.0 - x2
            den = jnp.maximum(1.0 - 2.0 * xy + x2 * y2, MIN_NORM)
            sn2 = jnp.maximum(
                (A * A * x2 + 2.0 * A * B * xy + B * B * y2) / (den * den),
                MIN_NORM)
            sn = _sqrt16(sn2)
            ratio = _artanh_ratio16(sn)
            kd = B * ratio / den
            p = kd * A
            q = kd * B

            dv = dbuf[pl.ds(g * 16, 16)]

            def _cdot(d16, carry):
                c = list(carry)
                d0 = d16 * 16
                wd16 = wv[0, pl.ds(d0, 16)]
                b116 = wv[1, pl.ds(d0, 16)]
                w216 = wv[2, pl.ds(d0, 16)]
                av = [plsc.load_gather(
                    tr, [rows, jnp.full((16,), D + d0 + j, _I32)])
                    for j in range(16)]
                bv = [plsc.load_gather(
                    tc, [rows, jnp.full((16,), D + d0 + j, _I32)])
                    for j in range(16)]
                for j in range(16):
                    pre = p * av[j] + q * bv[j] + (dv * wd16[j] + b116[j])
                    sig = 1.0 / (1.0 + jnp.exp(-pre))
                    k = j % 4
                    c[k] = c[k] + (pre * sig) * w216[j]
                return tuple(c)

            acc4 = lax.fori_loop(0, D // 16, _cdot, (zero16,) * 4)
            logit = (acc4[0] + acc4[1]) + (acc4[2] + acc4[3]) + b2
            em = mbuf[pl.ds(g * 16, 16)]
            att = em / (1.0 + jnp.exp(-logit))
            w1 = att * p
            w2 = att * q

            for j in range(16):
                e = g * 16 + j
                w1s = w1[j]
                w2s = w2[j]
                for i in range(D // 16):
                    vbuf[e, pl.ds(16 * i, 16)] = (
                        w1s * tr[e, pl.ds(16 * i, 16)]
                        + w2s * tc[e, pl.ds(16 * i, 16)])
            return 0

        lax.fori_loop(0, K // 16, _group, 0)
        pltpu.sync_copy(vbuf, spill_hbm.at[pl.ds(base, K)])

    _issue(0, slots[0])

    def _pair(tb, _):
        t0 = 2 * tb
        _issue(t0 + 1, slots[1])
        _wait(slots[0])
        _compute(t0, slots[0])
        _issue(t0 + 2, slots[0])
        _wait(slots[1])
        _compute(t0 + 1, slots[1])
        return 0

    lax.fori_loop(0, NPAIR, _pair, 0)
    _wait(slots[0])
    _compute(NCHUNK - 1, slots[0])


def _edges_sc(table, row, col, dist, emask, wvec):
    mesh = plsc.VectorSubcoreMesh(core_axis_name="c", subcore_axis_name="s")
    slot = [
        pltpu.VMEM((K,), _I32),
        pltpu.VMEM((K,), _I32),
        pltpu.VMEM((K,), _F32),
        pltpu.VMEM((K,), _F32),
        pltpu.VMEM((K, 2 * D), _F32),
        pltpu.VMEM((K, 2 * D), _F32),
    ]
    return pl.kernel(
        _edges_body,
        out_type=jax.ShapeDtypeStruct((E, D), jnp.float32),
        mesh=mesh,
        compiler_params=pltpu.CompilerParams(needs_layout_passes=False,
                                             use_tc_tiling_on_sc=False),
        scratch_types=slot + slot + [
            pltpu.VMEM((K, D), _F32),
            pltpu.VMEM((4, D), _F32),
            pltpu.SemaphoreType.DMA,
            pltpu.SemaphoreType.DMA,
        ],
    )(table, row, col, dist, emask, wvec)


def _scatter_body(spill_hbm, row_hbm, out_hbm, idxr, idxl, vbuf, acc, sem):
    cid = lax.axis_index("c")
    sid = lax.axis_index("s")
    lo = cid * HALF

    zero16 = jnp.zeros((16,), _F32)

    def _zrow(i, _):
        for j in range(D // 16):
            vbuf[i, pl.ds(16 * j, 16)] = zero16
        return 0

    lax.fori_loop(0, SK, _zrow, 0)
    for j, h in ((0, 128), (128, 128), (256, 72)):
        pltpu.sync_copy(vbuf.at[pl.ds(0, h)],
                        acc.at[pl.ds(sid * 328 + j, h)])
    plsc.subcore_barrier()

    nt = jnp.where(sid < SCHUNK_REM, SCHUNK_BASE + 1, SCHUNK_BASE)
    start = sid * SCHUNK_BASE + jnp.minimum(sid, SCHUNK_REM)

    def _chunk(t, _):
        base = (start + t) * SK
        pltpu.sync_copy(row_hbm.at[pl.ds(base, SK)], idxr)
        c1 = pltpu.async_copy(spill_hbm.at[pl.ds(base, SK)], vbuf, sem)
        for g in range(SK // 16):
            r = idxr[pl.ds(g * 16, 16)]
            rl = r - lo
            ok = (rl >= 0) & (rl < HALF)
            rl = jnp.where(ok, rl, HALF)
            idxl[pl.ds(g * 16, 16)] = rl
        c1.wait()
        pltpu.sync_copy(vbuf, acc.at[idxl], add=True)
        return 0

    lax.fori_loop(0, nt, _chunk, 0)
    plsc.subcore_barrier()
    pltpu.sync_copy(acc.at[pl.ds(sid * 320, 320)],
                    out_hbm.at[cid, pl.ds(sid * 320, 320)])


def _scatter_sc(spill, row):
    mesh = plsc.VectorSubcoreMesh(core_axis_name="c", subcore_axis_name="s")
    return pl.kernel(
        _scatter_body,
        out_type=jax.ShapeDtypeStruct((NC, HALF, D), jnp.float32),
        mesh=mesh,
        compiler_params=pltpu.CompilerParams(needs_layout_passes=False,
                                             use_tc_tiling_on_sc=False),
        scratch_types=[
            pltpu.VMEM((SK,), _I32),
            pltpu.VMEM((SK,), _I32),
            pltpu.VMEM((SK, D), _F32),
            pltpu.VMEM_SHARED((ACC_ROWS, D), _F32),
            pltpu.SemaphoreType.DMA,
        ],
    )(spill, row)


def kernel(x, distances, edges, node_mask, edge_mask, W_att1, b_att1,
           W_att2, b_att2, W_n1, b_n1, W_n2, b_n2):
    table = _stage1(x, W_att1[:D])
    wvec = jnp.stack([W_att1[2 * D], b_att1, W_att2[:, 0],
                      jnp.full((D,), b_att2[0], jnp.float32)])
    row = edges[0].astype(jnp.int32)
    col = edges[1].astype(jnp.int32)
    spill = _edges_sc(table, row, col, distances[:, 0], edge_mask[:, 0],
                      wvec)
    parts = _scatter_sc(spill, row)
    agg = jnp.concatenate([parts[0], parts[1]], axis=0)[:N]
    return _stage3(agg, x, W_n1[D:], b_n1, W_n2, b_n2)


# speedup vs baseline: 2.8928x; 1.7109x over previous
import jax
import jax.numpy as jnp
from jax import lax
from jax.experimental import pallas as pl
from jax.experimental.pallas import tpu as pltpu
from jax.experimental.pallas import tpu_sc as plsc

N = 10000
E = 320000
D = 128
MIN_NORM = 1e-15

NC = 2
NS = 16
NW = NC * NS
EPT = E // NW
K = 80
NCHUNK = EPT // K
NPAIR = (NCHUNK - 1) // 2
SK = 128
SNCHUNK = E // SK
SCHUNK_BASE = SNCHUNK // NS
SCHUNK_REM = SNCHUNK - SCHUNK_BASE * NS
HALF = 5120
ACC_ROWS = 5248

_F32 = jnp.float32
_I32 = jnp.int32


def _stage1_body(x_ref, w_ref, t_ref):
    t_ref[:, :D] = x_ref[...]
    t_ref[:, D:] = jnp.dot(x_ref[...], w_ref[...],
                           preferred_element_type=jnp.float32)


def _stage1(x, w1a):
    bm = 2000
    return pl.pallas_call(
        _stage1_body,
        grid=(N // bm,),
        in_specs=[
            pl.BlockSpec((bm, D), lambda i: (i, 0)),
            pl.BlockSpec((D, D), lambda i: (0, 0)),
        ],
        out_specs=pl.BlockSpec((bm, 2 * D), lambda i: (i, 0)),
        out_shape=jax.ShapeDtypeStruct((N, 2 * D), jnp.float32),
    )(x, w1a)


def _stage3_body(agg_ref, x_ref, wn1_ref, bn1_ref, wn2_ref, bn2_ref, o_ref):
    agg = agg_ref[...] * 0.01
    h2 = jnp.maximum(
        jnp.dot(agg, wn1_ref[...], preferred_element_type=jnp.float32)
        + bn1_ref[...], 0.0)
    s = (jnp.dot(h2, wn2_ref[...], preferred_element_type=jnp.float32)
         + bn2_ref[...])
    x = x_ref[...]
    u2 = jnp.sum(s * s, axis=-1, keepdims=True)
    u_norm = jnp.sqrt(jnp.clip(u2, MIN_NORM, None))
    x2 = jnp.sum(x * x, axis=-1, keepdims=True)
    lam = 2.0 / jnp.clip(1.0 - x2, MIN_NORM, None)
    second = jnp.tanh(0.5 * lam * u_norm) / u_norm * s
    y2 = jnp.sum(second * second, axis=-1, keepdims=True)
    xy = jnp.sum(x * second, axis=-1, keepdims=True)
    num = (1.0 + 2.0 * xy + y2) * x + (1.0 - x2) * second
    den = jnp.clip(1.0 + 2.0 * xy + x2 * y2, MIN_NORM, None)
    res = num / den
    rn = jnp.sqrt(jnp.clip(jnp.sum(res * res, axis=-1, keepdims=True),
                           MIN_NORM, None))
    maxnorm = 1.0 - 1e-5
    o_ref[...] = jnp.where(rn > maxnorm, res / rn * maxnorm, res)


def _stage3(agg, x, wn1b, bn1, wn2, bn2):
    bm = 2000
    return pl.pallas_call(
        _stage3_body,
        grid=(N // bm,),
        in_specs=[
            pl.BlockSpec((bm, D), lambda i: (i, 0)),
            pl.BlockSpec((bm, D), lambda i: (i, 0)),
            pl.BlockSpec((D, D), lambda i: (0, 0)),
            pl.BlockSpec((D,), lambda i: (0,)),
            pl.BlockSpec((D, D), lambda i: (0, 0)),
            pl.BlockSpec((D,), lambda i: (0,)),
        ],
        out_specs=pl.BlockSpec((bm, D), lambda i: (i, 0)),
        out_shape=jax.ShapeDtypeStruct((N, D), jnp.float32),
    )(agg, x, wn1b, bn1, wn2, bn2)


def _sqrt16(v):
    i = plsc.bitcast(v, _I32)
    i = jnp.int32(0x5F3759DF) - (i >> 1)
    y = plsc.bitcast(i, _F32)
    y = y * (1.5 - 0.5 * v * y * y)
    y = y * (1.5 - 0.5 * v * y * y)
    y = y * (1.5 - 0.5 * v * y * y)
    return v * y


def _artanh_ratio16(sn):
    z = jnp.minimum(sn, 1.0 - 1e-7)
    zz = z * z
    small = 1.0 + zz * (1.0 / 3.0 + zz * (0.2 + zz * (1.0 / 7.0
                                                      + zz * (1.0 / 9.0))))
    w = (1.0 + z) / (1.0 - z)
    iw = plsc.bitcast(w, _I32)
    ef = ((iw >> 23) - 127).astype(_F32)
    m = plsc.bitcast((iw & jnp.int32(0x007FFFFF)) | jnp.int32(0x3F800000),
                     _F32)
    s = (m - 1.0) / (m + 1.0)
    s2 = s * s
    log2m = s * (2.885390082 + s2 * (0.961796694 + s2 * (
        0.577078016 + s2 * (0.412198583 + s2 * 0.320598898))))
    big = 0.34657359028 * (ef + log2m) / sn
    return jnp.where(z < 0.25, small, big)


def _edges_body(t_hbm, row_hbm, col_hbm, dist_hbm, mask_hbm, wv_hbm,
                spill_hbm,
                idxr0, idxc0, dbuf0, mbuf0, tr0, tc0,
                idxr1, idxc1, dbuf1, mbuf1, tr1, tc1,
                vbuf, wv, sem0, sem1):
    cid = lax.axis_index("c")
    sid = lax.axis_index("s")
    w = cid * NS + sid
    ebase = w * EPT

    pltpu.sync_copy(wv_hbm, wv)
    b2 = wv[3, pl.ds(0, 16)][0]
    lanes = lax.iota(_I32, 16)
    zero16 = jnp.zeros((16,), _F32)

    slots = ((idxr0, idxc0, dbuf0, mbuf0, tr0, tc0, sem0),
             (idxr1, idxc1, dbuf1, mbuf1, tr1, tc1, sem1))

    def _issue(t, s):
        idxr, idxc, dbuf, mbuf, tr, tc, sem = s
        base = ebase + t * K
        pltpu.sync_copy(row_hbm.at[pl.ds(base, K)], idxr)
        pltpu.sync_copy(col_hbm.at[pl.ds(base, K)], idxc)
        pltpu.async_copy(dist_hbm.at[pl.ds(base, K)], dbuf.at[pl.ds(0, K)], sem)
        pltpu.async_copy(mask_hbm.at[pl.ds(base, K)], mbuf.at[pl.ds(0, K)], sem)
        pltpu.async_copy(t_hbm.at[idxr], tr, sem)
        pltpu.async_copy(t_hbm.at[idxc], tc, sem)

    def _wait(s):
        idxr, idxc, dbuf, mbuf, tr, tc, sem = s
        pltpu.make_async_copy(dist_hbm.at[pl.ds(0, K)], dbuf.at[pl.ds(0, K)], sem).wait()
        pltpu.make_async_copy(mask_hbm.at[pl.ds(0, K)], mbuf.at[pl.ds(0, K)], sem).wait()
        pltpu.make_async_copy(t_hbm.at[idxr], tr, sem).wait()
        pltpu.make_async_copy(t_hbm.at[idxc], tc, sem).wait()

    NB = D // 16
    wdv = [wv[0, pl.ds(16 * i, 16)] for i in range(NB)]
    b1v = [wv[1, pl.ds(16 * i, 16)] for i in range(NB)]
    w2v = [wv[2, pl.ds(16 * i, 16)] for i in range(NB)]
    masks8 = [lanes == j for j in range(8)]

    def _asm(vals):
        v = zero16
        for j in range(8):
            v = jnp.where(masks8[j], vals[j], v)
        return v

    def _compute(t, s):
        idxr, idxc, dbuf, mbuf, tr, tc, sem = s
        base = ebase + t * K

        def _group(g, _):
            exy = []
            ex2 = []
            ey2 = []
            for j in range(8):
                e = g * 8 + j
                sxy = zero16
                sx2 = zero16
                sy2 = zero16
                for i in range(NB):
                    a = tr[e, pl.ds(16 * i, 16)]
                    b = tc[e, pl.ds(16 * i, 16)]
                    sxy = sxy + a * b
                    sx2 = sx2 + a * a
                    sy2 = sy2 + b * b
                exy.append(jnp.sum(sxy))
                ex2.append(jnp.sum(sx2))
                ey2.append(jnp.sum(sy2))
            xy = _asm(exy)
            x2 = _asm(ex2)
            y2 = _asm(ey2)

            A = 2.0 * xy - 1.0 - y2
            B = 1.0 - x2
            den = jnp.maximum(1.0 - 2.0 * xy + x2 * y2, MIN_NORM)
            sn2 = jnp.maximum(
                (A * A * x2 + 2.0 * A * B * xy + B * B * y2) / (den * den),
                MIN_NORM)
            sn = _sqrt16(sn2)
            ratio = _artanh_ratio16(sn)
            kd = B * ratio / den
            p = kd * A
            q = kd * B

            dv = dbuf[pl.ds(g * 8, 16)]
            logits = []
            for j in range(8):
                e = g * 8 + j
                pj = p[j]
                qj = q[j]
                ddj = dv[j]
                la = zero16
                lb = zero16
                for i in range(NB):
                    gr = tr[e, pl.ds(D + 16 * i, 16)]
                    gc = tc[e, pl.ds(D + 16 * i, 16)]
                    pre = pj * gr + qj * gc + (ddj * wdv[i] + b1v[i])
                    sig = 1.0 / (1.0 + jnp.exp(-pre))
                    if i % 2 == 0:
                        la = la + (pre * sig) * w2v[i]
                    else:
                        lb = lb + (pre * sig) * w2v[i]
                logits.append(jnp.sum(la + lb))
            logit = _asm(logits) + b2
            em = mbuf[pl.ds(g * 8, 16)]
            att = em / (1.0 + jnp.exp(-logit))
            w1 = att * p
            w2 = att * q

            for j in range(8):
                e = g * 8 + j
                w1s = w1[j]
                w2s = w2[j]
                for i in range(NB):
                    vbuf[e, pl.ds(16 * i, 16)] = (
                        w1s * tr[e, pl.ds(16 * i, 16)]
                        + w2s * tc[e, pl.ds(16 * i, 16)])
            return 0

        lax.fori_loop(0, K // 8, _group, 0)
        pltpu.sync_copy(vbuf, spill_hbm.at[pl.ds(base, K)])

    _issue(0, slots[0])

    def _pair(tb, _):
        t0 = 2 * tb
        _issue(t0 + 1, slots[1])
        _wait(slots[0])
        _compute(t0, slots[0])
        _issue(t0 + 2, slots[0])
        _wait(slots[1])
        _compute(t0 + 1, slots[1])
        return 0

    lax.fori_loop(0, NPAIR, _pair, 0)
    _wait(slots[0])
    _compute(NCHUNK - 1, slots[0])


def _edges_sc(table, row, col, dist, emask, wvec):
    mesh = plsc.VectorSubcoreMesh(core_axis_name="c", subcore_axis_name="s")
    slot = [
        pltpu.VMEM((K,), _I32),
        pltpu.VMEM((K,), _I32),
        pltpu.VMEM((K + 16,), _F32),
        pltpu.VMEM((K + 16,), _F32),
        pltpu.VMEM((K, 2 * D), _F32),
        pltpu.VMEM((K, 2 * D), _F32),
    ]
    return pl.kernel(
        _edges_body,
        out_type=jax.ShapeDtypeStruct((E, D), jnp.float32),
        mesh=mesh,
        compiler_params=pltpu.CompilerParams(needs_layout_passes=False,
                                             use_tc_tiling_on_sc=False),
        scratch_types=slot + slot + [
            pltpu.VMEM((K, D), _F32),
            pltpu.VMEM((4, D), _F32),
            pltpu.SemaphoreType.DMA,
            pltpu.SemaphoreType.DMA,
        ],
    )(table, row, col, dist, emask, wvec)


def _scatter_body(spill_hbm, row_hbm, out_hbm, idxr, idxl, vbuf, acc, sem):
    cid = lax.axis_index("c")
    sid = lax.axis_index("s")
    lo = cid * HALF

    zero16 = jnp.zeros((16,), _F32)

    def _zrow(i, _):
        for j in range(D // 16):
            vbuf[i, pl.ds(16 * j, 16)] = zero16
        return 0

    lax.fori_loop(0, SK, _zrow, 0)
    for j, h in ((0, 128), (128, 128), (256, 72)):
        pltpu.sync_copy(vbuf.at[pl.ds(0, h)],
                        acc.at[pl.ds(sid * 328 + j, h)])
    plsc.subcore_barrier()

    nt = jnp.where(sid < SCHUNK_REM, SCHUNK_BASE + 1, SCHUNK_BASE)
    start = sid * SCHUNK_BASE + jnp.minimum(sid, SCHUNK_REM)

    def _chunk(t, _):
        base = (start + t) * SK
        pltpu.sync_copy(row_hbm.at[pl.ds(base, SK)], idxr)
        c1 = pltpu.async_copy(spill_hbm.at[pl.ds(base, SK)], vbuf, sem)
        for g in range(SK // 16):
            r = idxr[pl.ds(g * 16, 16)]
            rl = r - lo
            ok = (rl >= 0) & (rl < HALF)
            rl = jnp.where(ok, rl, HALF)
            idxl[pl.ds(g * 16, 16)] = rl
        c1.wait()
        pltpu.sync_copy(vbuf, acc.at[idxl], add=True)
        return 0

    lax.fori_loop(0, nt, _chunk, 0)
    plsc.subcore_barrier()
    pltpu.sync_copy(acc.at[pl.ds(sid * 320, 320)],
                    out_hbm.at[cid, pl.ds(sid * 320, 320)])


def _scatter_sc(spill, row):
    mesh = plsc.VectorSubcoreMesh(core_axis_name="c", subcore_axis_name="s")
    return pl.kernel(
        _scatter_body,
        out_type=jax.ShapeDtypeStruct((NC, HALF, D), jnp.float32),
        mesh=mesh,
        compiler_params=pltpu.CompilerParams(needs_layout_passes=False,
                                             use_tc_tiling_on_sc=False),
        scratch_types=[
            pltpu.VMEM((SK,), _I32),
            pltpu.VMEM((SK,), _I32),
            pltpu.VMEM((SK, D), _F32),
            pltpu.VMEM_SHARED((ACC_ROWS, D), _F32),
            pltpu.SemaphoreType.DMA,
        ],
    )(spill, row)


def kernel(x, distances, edges, node_mask, edge_mask, W_att1, b_att1,
           W_att2, b_att2, W_n1, b_n1, W_n2, b_n2):
    table = _stage1(x, W_att1[:D])
    wvec = jnp.stack([W_att1[2 * D], b_att1, W_att2[:, 0],
                      jnp.full((D,), b_att2[0], jnp.float32)])
    row = edges[0].astype(jnp.int32)
    col = edges[1].astype(jnp.int32)
    spill = _edges_sc(table, row, col, distances[:, 0], edge_mask[:, 0],
                      wvec)
    parts = _scatter_sc(spill, row)
    agg = jnp.concatenate([parts[0], parts[1]], axis=0)[:N]
    return _stage3(agg, x, W_n1[D:], b_n1, W_n2, b_n2)


# speedup vs baseline: 3.0660x; 1.0599x over previous
import jax
import jax.numpy as jnp
from jax import lax
from jax.experimental import pallas as pl
from jax.experimental.pallas import tpu as pltpu
from jax.experimental.pallas import tpu_sc as plsc

N = 10000
E = 320000
D = 128
MIN_NORM = 1e-15

NC = 2
NS = 16
NW = NC * NS
EPT = E // NW
K = 80
NCHUNK = EPT // K
NPAIR = (NCHUNK - 1) // 2
SK = 80
SCHUNKS_PT = E // SK // NS
HALF = 5120
ACC_ROWS = 5248

_F32 = jnp.float32
_I32 = jnp.int32


def _stage1_body(x_ref, w_ref, t_ref):
    t_ref[:, :D] = x_ref[...]
    t_ref[:, D:] = jnp.dot(x_ref[...], w_ref[...],
                           preferred_element_type=jnp.float32)


def _stage1(x, w1a):
    bm = 2000
    return pl.pallas_call(
        _stage1_body,
        grid=(N // bm,),
        in_specs=[
            pl.BlockSpec((bm, D), lambda i: (i, 0)),
            pl.BlockSpec((D, D), lambda i: (0, 0)),
        ],
        out_specs=pl.BlockSpec((bm, 2 * D), lambda i: (i, 0)),
        out_shape=jax.ShapeDtypeStruct((N, 2 * D), jnp.float32),
    )(x, w1a)


def _stage3_body(agg_ref, x_ref, wn1_ref, bn1_ref, wn2_ref, bn2_ref, o_ref):
    agg = agg_ref[...] * 0.01
    h2 = jnp.maximum(
        jnp.dot(agg, wn1_ref[...], preferred_element_type=jnp.float32)
        + bn1_ref[...], 0.0)
    s = (jnp.dot(h2, wn2_ref[...], preferred_element_type=jnp.float32)
         + bn2_ref[...])
    x = x_ref[...]
    u2 = jnp.sum(s * s, axis=-1, keepdims=True)
    u_norm = jnp.sqrt(jnp.clip(u2, MIN_NORM, None))
    x2 = jnp.sum(x * x, axis=-1, keepdims=True)
    lam = 2.0 / jnp.clip(1.0 - x2, MIN_NORM, None)
    second = jnp.tanh(0.5 * lam * u_norm) / u_norm * s
    y2 = jnp.sum(second * second, axis=-1, keepdims=True)
    xy = jnp.sum(x * second, axis=-1, keepdims=True)
    num = (1.0 + 2.0 * xy + y2) * x + (1.0 - x2) * second
    den = jnp.clip(1.0 + 2.0 * xy + x2 * y2, MIN_NORM, None)
    res = num / den
    rn = jnp.sqrt(jnp.clip(jnp.sum(res * res, axis=-1, keepdims=True),
                           MIN_NORM, None))
    maxnorm = 1.0 - 1e-5
    o_ref[...] = jnp.where(rn > maxnorm, res / rn * maxnorm, res)


def _stage3(agg, x, wn1b, bn1, wn2, bn2):
    bm = 2000
    return pl.pallas_call(
        _stage3_body,
        grid=(N // bm,),
        in_specs=[
            pl.BlockSpec((bm, D), lambda i: (i, 0)),
            pl.BlockSpec((bm, D), lambda i: (i, 0)),
            pl.BlockSpec((D, D), lambda i: (0, 0)),
            pl.BlockSpec((D,), lambda i: (0,)),
            pl.BlockSpec((D, D), lambda i: (0, 0)),
            pl.BlockSpec((D,), lambda i: (0,)),
        ],
        out_specs=pl.BlockSpec((bm, D), lambda i: (i, 0)),
        out_shape=jax.ShapeDtypeStruct((N, D), jnp.float32),
    )(agg, x, wn1b, bn1, wn2, bn2)


def _sqrt16(v):
    i = plsc.bitcast(v, _I32)
    i = jnp.int32(0x5F3759DF) - (i >> 1)
    y = plsc.bitcast(i, _F32)
    y = y * (1.5 - 0.5 * v * y * y)
    y = y * (1.5 - 0.5 * v * y * y)
    y = y * (1.5 - 0.5 * v * y * y)
    return v * y


def _artanh_ratio16(sn):
    z = jnp.minimum(sn, 1.0 - 1e-7)
    zz = z * z
    small = 1.0 + zz * (1.0 / 3.0 + zz * (0.2 + zz * (1.0 / 7.0
                                                      + zz * (1.0 / 9.0))))
    w = (1.0 + z) / (1.0 - z)
    iw = plsc.bitcast(w, _I32)
    ef = ((iw >> 23) - 127).astype(_F32)
    m = plsc.bitcast((iw & jnp.int32(0x007FFFFF)) | jnp.int32(0x3F800000),
                     _F32)
    s = (m - 1.0) / (m + 1.0)
    s2 = s * s
    log2m = s * (2.885390082 + s2 * (0.961796694 + s2 * (
        0.577078016 + s2 * (0.412198583 + s2 * 0.320598898))))
    big = 0.34657359028 * (ef + log2m) / sn
    return jnp.where(z < 0.25, small, big)


def _edges_body(t_hbm, row_hbm, col_hbm, dist_hbm, mask_hbm, wv_hbm,
                spill_hbm,
                idxr0, idxc0, dbuf0, mbuf0, tr0, tc0,
                idxr1, idxc1, dbuf1, mbuf1, tr1, tc1,
                vbuf, wv, sem0, sem1):
    cid = lax.axis_index("c")
    sid = lax.axis_index("s")
    w = cid * NS + sid
    ebase = w * EPT

    pltpu.sync_copy(wv_hbm, wv)
    b2 = wv[3, pl.ds(0, 16)][0]
    lanes = lax.iota(_I32, 16)
    zero16 = jnp.zeros((16,), _F32)

    slots = ((idxr0, idxc0, dbuf0, mbuf0, tr0, tc0, sem0),
             (idxr1, idxc1, dbuf1, mbuf1, tr1, tc1, sem1))

    def _issue(t, s):
        idxr, idxc, dbuf, mbuf, tr, tc, sem = s
        base = ebase + t * K
        pltpu.sync_copy(row_hbm.at[pl.ds(base, K)], idxr)
        pltpu.sync_copy(col_hbm.at[pl.ds(base, K)], idxc)
        pltpu.async_copy(dist_hbm.at[pl.ds(base, K)], dbuf.at[pl.ds(0, K)], sem)
        pltpu.async_copy(mask_hbm.at[pl.ds(base, K)], mbuf.at[pl.ds(0, K)], sem)
        pltpu.async_copy(t_hbm.at[idxr], tr, sem)
        pltpu.async_copy(t_hbm.at[idxc], tc, sem)

    def _wait(s):
        idxr, idxc, dbuf, mbuf, tr, tc, sem = s
        pltpu.make_async_copy(dist_hbm.at[pl.ds(0, K)], dbuf.at[pl.ds(0, K)], sem).wait()
        pltpu.make_async_copy(mask_hbm.at[pl.ds(0, K)], mbuf.at[pl.ds(0, K)], sem).wait()
        pltpu.make_async_copy(t_hbm.at[idxr], tr, sem).wait()
        pltpu.make_async_copy(t_hbm.at[idxc], tc, sem).wait()

    NB = D // 16
    wdv = [wv[0, pl.ds(16 * i, 16)] for i in range(NB)]
    b1v = [wv[1, pl.ds(16 * i, 16)] for i in range(NB)]
    w2v = [wv[2, pl.ds(16 * i, 16)] for i in range(NB)]
    masks8 = [lanes == j for j in range(8)]

    def _asm(vals):
        v = zero16
        for j in range(8):
            v = jnp.where(masks8[j], vals[j], v)
        return v

    def _compute(t, s):
        idxr, idxc, dbuf, mbuf, tr, tc, sem = s
        base = ebase + t * K

        def _group(g, _):
            exy = []
            ex2 = []
            ey2 = []
            for j in range(8):
                e = g * 8 + j
                sxy = zero16
                sx2 = zero16
                sy2 = zero16
                for i in range(NB):
                    a = tr[e, pl.ds(16 * i, 16)]
                    b = tc[e, pl.ds(16 * i, 16)]
                    sxy = sxy + a * b
                    sx2 = sx2 + a * a
                    sy2 = sy2 + b * b
                exy.append(jnp.sum(sxy))
                ex2.append(jnp.sum(sx2))
                ey2.append(jnp.sum(sy2))
            xy = _asm(exy)
            x2 = _asm(ex2)
            y2 = _asm(ey2)

            A = 2.0 * xy - 1.0 - y2
            B = 1.0 - x2
            den = jnp.maximum(1.0 - 2.0 * xy + x2 * y2, MIN_NORM)
            sn2 = jnp.maximum(
                (A * A * x2 + 2.0 * A * B * xy + B * B * y2) / (den * den),
                MIN_NORM)
            sn = _sqrt16(sn2)
            ratio = _artanh_ratio16(sn)
            kd = B * ratio / den
            p = kd * A
            q = kd * B

            dv = dbuf[pl.ds(g * 8, 16)]
            logits = []
            for j in range(8):
                e = g * 8 + j
                pj = p[j]
                qj = q[j]
                ddj = dv[j]
                la = zero16
                lb = zero16
                for i in range(NB):
                    gr = tr[e, pl.ds(D + 16 * i, 16)]
                    gc = tc[e, pl.ds(D + 16 * i, 16)]
                    pre = pj * gr + qj * gc + (ddj * wdv[i] + b1v[i])
                    sig = 1.0 / (1.0 + jnp.exp(-pre))
                    if i % 2 == 0:
                        la = la + (pre * sig) * w2v[i]
                    else:
                        lb = lb + (pre * sig) * w2v[i]
                logits.append(jnp.sum(la + lb))
            logit = _asm(logits) + b2
            em = mbuf[pl.ds(g * 8, 16)]
            att = em / (1.0 + jnp.exp(-logit))
            w1 = att * p
            w2 = att * q

            for j in range(8):
                e = g * 8 + j
                w1s = w1[j]
                w2s = w2[j]
                for i in range(NB):
                    vbuf[e, pl.ds(16 * i, 16)] = (
                        w1s * tr[e, pl.ds(16 * i, 16)]
                        + w2s * tc[e, pl.ds(16 * i, 16)])
            return 0

        lax.fori_loop(0, K // 8, _group, 0)
        pltpu.sync_copy(vbuf, spill_hbm.at[pl.ds(base, K)])

    _issue(0, slots[0])

    def _pair(tb, _):
        t0 = 2 * tb
        _issue(t0 + 1, slots[1])
        _wait(slots[0])
        _compute(t0, slots[0])
        _issue(t0 + 2, slots[0])
        _wait(slots[1])
        _compute(t0 + 1, slots[1])
        return 0

    lax.fori_loop(0, NPAIR, _pair, 0)
    _wait(slots[0])
    _compute(NCHUNK - 1, slots[0])


def _edges_sc(table, row, col, dist, emask, wvec):
    mesh = plsc.VectorSubcoreMesh(core_axis_name="c", subcore_axis_name="s")
    slot = [
        pltpu.VMEM((K,), _I32),
        pltpu.VMEM((K,), _I32),
        pltpu.VMEM((K + 16,), _F32),
        pltpu.VMEM((K + 16,), _F32),
        pltpu.VMEM((K, 2 * D), _F32),
        pltpu.VMEM((K, 2 * D), _F32),
    ]
    return pl.kernel(
        _edges_body,
        out_type=jax.ShapeDtypeStruct((E, D), jnp.float32),
        mesh=mesh,
        compiler_params=pltpu.CompilerParams(needs_layout_passes=False,
                                             use_tc_tiling_on_sc=False),
        scratch_types=slot + slot + [
            pltpu.VMEM((K, D), _F32),
            pltpu.VMEM((4, D), _F32),
            pltpu.SemaphoreType.DMA,
            pltpu.SemaphoreType.DMA,
        ],
    )(table, row, col, dist, emask, wvec)


def _scatter_body(spill_hbm, row_hbm, out_hbm, idxr0, idxl0, vbuf0, sem0,
                  idxr1, idxl1, vbuf1, sem1, acc):
    cid = lax.axis_index("c")
    sid = lax.axis_index("s")
    lo = cid * HALF

    zero16 = jnp.zeros((16,), _F32)

    def _zrow(i, _):
        for j in range(D // 16):
            vbuf0[i, pl.ds(16 * j, 16)] = zero16
        return 0

    lax.fori_loop(0, SK, _zrow, 0)
    for j in range(5):
        h = 80 if j < 4 else 8
        pltpu.sync_copy(vbuf0.at[pl.ds(0, h)],
                        acc.at[pl.ds(sid * 328 + j * 80, h)])
    plsc.subcore_barrier()

    start = sid * SCHUNKS_PT
    slots = ((idxr0, idxl0, vbuf0, sem0), (idxr1, idxl1, vbuf1, sem1))

    def _issue(t, s):
        idxr, idxl, vbuf, sem = s
        base = (start + t) * SK
        pltpu.sync_copy(row_hbm.at[pl.ds(base, SK)], idxr)
        pltpu.async_copy(spill_hbm.at[pl.ds(base, SK)], vbuf, sem)

    def _process(t, s):
        idxr, idxl, vbuf, sem = s
        for g in range(SK // 16):
            r = idxr[pl.ds(g * 16, 16)]
            rl = r - lo
            ok = (rl >= 0) & (rl < HALF)
            rl = jnp.where(ok, rl, HALF)
            idxl[pl.ds(g * 16, 16)] = rl
        pltpu.make_async_copy(spill_hbm.at[pl.ds(0, SK)], vbuf, sem).wait()
        pltpu.sync_copy(vbuf, acc.at[idxl], add=True)

    _issue(0, slots[0])

    def _pair(tb, _):
        t0 = 2 * tb
        _issue(t0 + 1, slots[1])
        _process(t0, slots[0])
        _issue(t0 + 2, slots[0])
        _process(t0 + 1, slots[1])
        return 0

    lax.fori_loop(0, (SCHUNKS_PT - 2) // 2, _pair, 0)
    _issue(SCHUNKS_PT - 1, slots[1])
    _process(SCHUNKS_PT - 2, slots[0])
    _process(SCHUNKS_PT - 1, slots[1])
    plsc.subcore_barrier()
    pltpu.sync_copy(acc.at[pl.ds(sid * 320, 320)],
                    out_hbm.at[cid, pl.ds(sid * 320, 320)])


def _scatter_sc(spill, row):
    mesh = plsc.VectorSubcoreMesh(core_axis_name="c", subcore_axis_name="s")
    slot = [
        pltpu.VMEM((SK,), _I32),
        pltpu.VMEM((SK,), _I32),
        pltpu.VMEM((SK, D), _F32),
        pltpu.SemaphoreType.DMA,
    ]
    return pl.kernel(
        _scatter_body,
        out_type=jax.ShapeDtypeStruct((NC, HALF, D), jnp.float32),
        mesh=mesh,
        compiler_params=pltpu.CompilerParams(needs_layout_passes=False,
                                             use_tc_tiling_on_sc=False),
        scratch_types=slot + slot + [
            pltpu.VMEM_SHARED((ACC_ROWS, D), _F32),
        ],
    )(spill, row)


def kernel(x, distances, edges, node_mask, edge_mask, W_att1, b_att1,
           W_att2, b_att2, W_n1, b_n1, W_n2, b_n2):
    table = _stage1(x, W_att1[:D])
    wvec = jnp.stack([W_att1[2 * D], b_att1, W_att2[:, 0],
                      jnp.full((D,), b_att2[0], jnp.float32)])
    row = edges[0].astype(jnp.int32)
    col = edges[1].astype(jnp.int32)
    spill = _edges_sc(table, row, col, distances[:, 0], edge_mask[:, 0],
                      wvec)
    parts = _scatter_sc(spill, row)
    agg = jnp.concatenate([parts[0], parts[1]], axis=0)[:N]
    return _stage3(agg, x, W_n1[D:], b_n1, W_n2, b_n2)
